# Initial kernel scaffold; baseline (speedup 1.0000x reference)
import jax
import jax.numpy as jnp
from jax.experimental import pallas as pl

_TOPK = 4
_EXTRA_FRAMES = (1, 3, 5, 7)
_SHARED_TEACHER = (2, 4, 6)
_SHARED_STUDENT = (1, 2, 3)
_EPS = 1e-8


def _loss_kernel(ref_t_ref, ref_s_ref, extra_ref, extra_t_ref,
                 sh_t_ref, sh_s_ref, out_ref):
    f32 = jnp.float32
    ref_t = ref_t_ref[...]
    ref_s = ref_s_ref[...]
    extra = extra_ref[...]
    extra_t = extra_t_ref[...]
    sh_t = sh_t_ref[...]
    sh_s = sh_s_ref[...]

    rt_norm = jnp.sqrt(jnp.sum(ref_t * ref_t, axis=1, keepdims=True))
    rtn = ref_t / jnp.maximum(rt_norm, _EPS)
    key_norm = jnp.sqrt(jnp.sum(extra_t * extra_t, axis=0, keepdims=True))
    keyn = extra_t / jnp.maximum(key_norm, _EPS)
    sim = jnp.dot(rtn, keyn, preferred_element_type=f32)

    lane = jax.lax.broadcasted_iota(jnp.int32, sim.shape, 1)
    work = sim
    idxs = []
    for _ in range(_TOPK):
        m = jnp.max(work, axis=1, keepdims=True)
        amax = jnp.min(jnp.where(work == m, lane, jnp.int32(sim.shape[1])),
                       axis=1, keepdims=True)
        idxs.append(amax)
        work = jnp.where(lane == amax, -jnp.inf, work)

    Nr_t = jnp.sum(ref_t * ref_t, axis=1, keepdims=True)
    Nr_s = jnp.sum(ref_s * ref_s, axis=1, keepdims=True)
    Nm_t = jnp.sum(sh_t * sh_t, axis=0, keepdims=True)
    Nm_s = jnp.sum(sh_s * sh_s, axis=0, keepdims=True)
    G1t = jnp.dot(ref_t, sh_t, preferred_element_type=f32)
    G1s = jnp.dot(ref_s, sh_s, preferred_element_type=f32)

    def _den(x2):
        return jnp.maximum(jnp.sqrt(jnp.maximum(x2, 0.0)), _EPS)

    d_u1t = _den(Nm_t - 2.0 * G1t + Nr_t)
    d_u1s = _den(Nm_s - 2.0 * G1s + Nr_s)

    acc = jnp.float32(0.0)
    for k in range(_TOPK):
        onehot = (lane == idxs[k]).astype(f32)
        sh = jnp.dot(onehot, extra, preferred_element_type=f32)
        Ns = jnp.sum(sh * sh, axis=1, keepdims=True)
        g2t = jnp.sum(ref_t * sh, axis=1, keepdims=True)
        g2s = jnp.sum(ref_s * sh, axis=1, keepdims=True)
        G3t = jnp.dot(sh, sh_t, preferred_element_type=f32)
        G3s = jnp.dot(sh, sh_s, preferred_element_type=f32)

        d_vt = _den(Ns - 2.0 * g2t + Nr_t)
        d_vs = _den(Ns - 2.0 * g2s + Nr_s)

        a1t = (G3t - G1t - g2t + Nr_t) / (d_u1t * d_vt)
        a1s = (G3s - G1s - g2s + Nr_s) / (d_u1s * d_vs)
        acc = acc + jnp.sum(jnp.abs(a1s - a1t))

        a2t = (G1t - G3t - g2t + Ns) / (d_vt * _den(Nm_t - 2.0 * G3t + Ns))
        a2s = (G1s - G3s - g2s + Ns) / (d_vs * _den(Nm_s - 2.0 * G3s + Ns))
        acc = acc + jnp.sum(jnp.abs(a2s - a2t))

        a3t = (g2t - G1t - G3t + Nm_t) / (d_u1t * _den(Ns - 2.0 * G3t + Nm_t))
        a3s = (g2s - G1s - G3s + Nm_s) / (d_u1s * _den(Ns - 2.0 * G3s + Nm_s))
        acc = acc + jnp.sum(jnp.abs(a3s - a3t))

    total = 64 * 192 * _TOPK
    out_ref[0, 0] = acc / jnp.float32(total)


def kernel(teacher_feats, student_feats, ref_perm, shared_perm):
    tf = jax.lax.stop_gradient(teacher_feats)[0]
    sf = student_feats[0]
    ref_t = tf[0][ref_perm]
    ref_s = sf[0][ref_perm]
    extra = jnp.concatenate([tf[e] for e in _EXTRA_FRAMES], axis=0)
    extra_t = extra.T
    sh_t = jnp.concatenate([tf[t][shared_perm] for t in _SHARED_TEACHER], 0).T
    sh_s = jnp.concatenate([sf[s][shared_perm] for s in _SHARED_STUDENT], 0).T

    out = pl.pallas_call(
        _loss_kernel,
        out_shape=jax.ShapeDtypeStruct((1, 1), jnp.float32),
    )(ref_t, ref_s, extra, extra_t, sh_t, sh_s)
    return out[0, 0]


# speedup vs baseline: 8.1162x; 8.1162x over previous
import jax
import jax.numpy as jnp
from jax.experimental import pallas as pl

_TOPK = 4
_EXTRA_FRAMES = (1, 3, 5, 7)
_SHARED_TEACHER = (2, 4, 6)
_SHARED_STUDENT = (1, 2, 3)
_EPS = 1e-8


def _loss_kernel(ref_t_ref, ref_s_ref, extra_ref, extra_t_ref,
                 sh_t_ref, sh_s_ref, out_ref):
    f32 = jnp.float32
    ref_t = ref_t_ref[...]
    ref_s = ref_s_ref[...]
    extra = extra_ref[...]
    extra_t = extra_t_ref[...]
    sh_t = sh_t_ref[...]
    sh_s = sh_s_ref[...]

    rt_norm = jnp.sqrt(jnp.sum(ref_t * ref_t, axis=1, keepdims=True))
    rtn = ref_t / jnp.maximum(rt_norm, _EPS)
    key_norm = jnp.sqrt(jnp.sum(extra_t * extra_t, axis=0, keepdims=True))
    keyn = extra_t / jnp.maximum(key_norm, _EPS)
    sim = jnp.dot(rtn, keyn, preferred_element_type=f32)

    lane = jax.lax.broadcasted_iota(jnp.int32, sim.shape, 1)
    work = sim
    idxs = []
    for _ in range(_TOPK):
        m = jnp.max(work, axis=1, keepdims=True)
        amax = jnp.min(jnp.where(work == m, lane, jnp.int32(sim.shape[1])),
                       axis=1, keepdims=True)
        idxs.append(amax)
        work = jnp.where(lane == amax, -jnp.inf, work)

    Nr_t = jnp.sum(ref_t * ref_t, axis=1, keepdims=True)
    Nr_s = jnp.sum(ref_s * ref_s, axis=1, keepdims=True)
    Nm_t = jnp.sum(sh_t * sh_t, axis=0, keepdims=True)
    Nm_s = jnp.sum(sh_s * sh_s, axis=0, keepdims=True)
    G1t = jnp.dot(ref_t, sh_t, preferred_element_type=f32)
    G1s = jnp.dot(ref_s, sh_s, preferred_element_type=f32)

    def _den(x2):
        return jnp.maximum(jnp.sqrt(jnp.maximum(x2, 0.0)), _EPS)

    d_u1t = _den(Nm_t - 2.0 * G1t + Nr_t)
    d_u1s = _den(Nm_s - 2.0 * G1s + Nr_s)

    acc = jnp.float32(0.0)
    for k in range(_TOPK):
        onehot = (lane == idxs[k]).astype(f32)
        sh = jnp.dot(onehot, extra, preferred_element_type=f32)
        Ns = jnp.sum(sh * sh, axis=1, keepdims=True)
        g2t = jnp.sum(ref_t * sh, axis=1, keepdims=True)
        g2s = jnp.sum(ref_s * sh, axis=1, keepdims=True)
        G3t = jnp.dot(sh, sh_t, preferred_element_type=f32)
        G3s = jnp.dot(sh, sh_s, preferred_element_type=f32)

        d_vt = _den(Ns - 2.0 * g2t + Nr_t)
        d_vs = _den(Ns - 2.0 * g2s + Nr_s)

        a1t = (G3t - G1t - g2t + Nr_t) / (d_u1t * d_vt)
        a1s = (G3s - G1s - g2s + Nr_s) / (d_u1s * d_vs)
        acc = acc + jnp.sum(jnp.abs(a1s - a1t))

        a2t = (G1t - G3t - g2t + Ns) / (d_vt * _den(Nm_t - 2.0 * G3t + Ns))
        a2s = (G1s - G3s - g2s + Ns) / (d_vs * _den(Nm_s - 2.0 * G3s + Ns))
        acc = acc + jnp.sum(jnp.abs(a2s - a2t))

        a3t = (g2t - G1t - G3t + Nm_t) / (d_u1t * _den(Ns - 2.0 * G3t + Nm_t))
        a3s = (g2s - G1s - G3s + Nm_s) / (d_u1s * _den(Ns - 2.0 * G3s + Nm_s))
        acc = acc + jnp.sum(jnp.abs(a3s - a3t))

    total = 64 * 192 * _TOPK
    out_ref[...] = jnp.broadcast_to(acc / jnp.float32(total), out_ref.shape)


def kernel(teacher_feats, student_feats, ref_perm, shared_perm):
    tf = jax.lax.stop_gradient(teacher_feats)[0]
    sf = student_feats[0]
    ref_t = tf[0][ref_perm]
    ref_s = sf[0][ref_perm]
    extra = jnp.concatenate([tf[e] for e in _EXTRA_FRAMES], axis=0)
    extra_t = extra.T
    sh_t = jnp.concatenate([tf[t][shared_perm] for t in _SHARED_TEACHER], 0).T
    sh_s = jnp.concatenate([sf[s][shared_perm] for s in _SHARED_STUDENT], 0).T

    out = pl.pallas_call(
        _loss_kernel,
        out_shape=jax.ShapeDtypeStruct((1, 1), jnp.float32),
    )(ref_t, ref_s, extra, extra_t, sh_t, sh_s)
    return out[0, 0]


# speedup vs baseline: 8.3400x; 1.0276x over previous
import jax
import jax.numpy as jnp
from jax.experimental import pallas as pl

_TOPK = 4
_EXTRA_FRAMES = (1, 3, 5, 7)
_SHARED_TEACHER = (2, 4, 6)
_SHARED_STUDENT = (1, 2, 3)
_EPS = 1e-8
_NREF = 64
_P = 1024
_D = 192


def _dotT(a, b):
    return jax.lax.dot_general(a, b, (((1,), (1,)), ((), ())),
                               preferred_element_type=jnp.float32)


def _dot(a, b):
    return jnp.dot(a, b, preferred_element_type=jnp.float32)


def _loss_kernel(tf_ref, sf_ref, rperm_ref, sperm_ref, out_ref):
    f32 = jnp.float32

    lane_p = jax.lax.broadcasted_iota(jnp.int32, (_NREF, _P), 1)
    ph_r = (lane_p == rperm_ref[...]).astype(f32)
    ph_s = (lane_p == sperm_ref[...]).astype(f32)

    ref_t = _dot(ph_r, tf_ref[0])
    ref_s = _dot(ph_r, sf_ref[0])
    sh_t = jnp.concatenate([_dot(ph_s, tf_ref[t]) for t in _SHARED_TEACHER], 0)
    sh_s = jnp.concatenate([_dot(ph_s, sf_ref[s]) for s in _SHARED_STUDENT], 0)

    Nr_t = jnp.sum(ref_t * ref_t, axis=1, keepdims=True)
    rtn = ref_t / jnp.maximum(jnp.sqrt(Nr_t), _EPS)
    sims = []
    for e in _EXTRA_FRAMES:
        frame = tf_ref[e]
        kn2 = jnp.sum(frame * frame, axis=1, keepdims=True)
        kn = frame / jnp.maximum(jnp.sqrt(kn2), _EPS)
        sims.append(_dotT(rtn, kn))
    sim = jnp.concatenate(sims, axis=1)

    lane = jax.lax.broadcasted_iota(jnp.int32, sim.shape, 1)
    work = sim
    idxs = []
    for _ in range(_TOPK):
        m = jnp.max(work, axis=1, keepdims=True)
        amax = jnp.min(jnp.where(work == m, lane, jnp.int32(sim.shape[1])),
                       axis=1, keepdims=True)
        idxs.append(amax)
        work = jnp.where(lane == amax, -jnp.inf, work)

    Nr_s = jnp.sum(ref_s * ref_s, axis=1, keepdims=True)
    ones_d = jnp.ones((1, _D), dtype=f32)
    Nm_t = _dotT(ones_d, sh_t * sh_t)
    Nm_s = _dotT(ones_d, sh_s * sh_s)
    G1t = _dotT(ref_t, sh_t)
    G1s = _dotT(ref_s, sh_s)

    def _den(x2):
        return jnp.maximum(jnp.sqrt(jnp.maximum(x2, 0.0)), _EPS)

    d_u1t = _den(Nm_t - 2.0 * G1t + Nr_t)
    d_u1s = _den(Nm_s - 2.0 * G1s + Nr_s)

    acc = jnp.float32(0.0)
    for k in range(_TOPK):
        onehot = (lane == idxs[k]).astype(f32)
        sh = jnp.zeros((_NREF, _D), dtype=f32)
        for i, e in enumerate(_EXTRA_FRAMES):
            sh = sh + _dot(onehot[:, i * _P:(i + 1) * _P], tf_ref[e])
        Ns = jnp.sum(sh * sh, axis=1, keepdims=True)
        g2t = jnp.sum(ref_t * sh, axis=1, keepdims=True)
        g2s = jnp.sum(ref_s * sh, axis=1, keepdims=True)
        G3t = _dotT(sh, sh_t)
        G3s = _dotT(sh, sh_s)

        d_vt = _den(Ns - 2.0 * g2t + Nr_t)
        d_vs = _den(Ns - 2.0 * g2s + Nr_s)

        a1t = (G3t - G1t - g2t + Nr_t) / (d_u1t * d_vt)
        a1s = (G3s - G1s - g2s + Nr_s) / (d_u1s * d_vs)
        acc = acc + jnp.sum(jnp.abs(a1s - a1t))

        a2t = (G1t - G3t - g2t + Ns) / (d_vt * _den(Nm_t - 2.0 * G3t + Ns))
        a2s = (G1s - G3s - g2s + Ns) / (d_vs * _den(Nm_s - 2.0 * G3s + Ns))
        acc = acc + jnp.sum(jnp.abs(a2s - a2t))

        a3t = (g2t - G1t - G3t + Nm_t) / (d_u1t * _den(Ns - 2.0 * G3t + Nm_t))
        a3s = (g2s - G1s - G3s + Nm_s) / (d_u1s * _den(Ns - 2.0 * G3s + Nm_s))
        acc = acc + jnp.sum(jnp.abs(a3s - a3t))

    total = _NREF * 192 * _TOPK
    out_ref[...] = jnp.broadcast_to(acc / jnp.float32(total), out_ref.shape)


def kernel(teacher_feats, student_feats, ref_perm, shared_perm):
    tf = jax.lax.stop_gradient(teacher_feats)[0]
    sf = student_feats[0]
    rperm = ref_perm.astype(jnp.int32).reshape(_NREF, 1)
    sperm = shared_perm.astype(jnp.int32).reshape(_NREF, 1)

    out = pl.pallas_call(
        _loss_kernel,
        out_shape=jax.ShapeDtypeStruct((1, 1), jnp.float32),
    )(tf, sf, rperm, sperm)
    return out[0, 0]
